# baseline (speedup 1.0000x reference)
import functools

import jax
import jax.numpy as jnp
from jax import lax
from jax.experimental import pallas as pl
from jax.experimental.pallas import tpu as pltpu
from jax.experimental.pallas import tpu_sc as plsc

VOCAB = 1_000_000
DIM = 64
BATCH = 16384
L = 50

ROWS_BLK = 8192
TC_GRID = pl.cdiv(VOCAB, ROWS_BLK)


def _matvec_body(x_ref, w_ref, b_ref, t_ref):
    t = jnp.dot(x_ref[...], w_ref[...], preferred_element_type=jnp.float32)
    t_ref[...] = t + b_ref[...] * (1.0 / L)


def _matvec(table, W, b):
    return pl.pallas_call(
        _matvec_body,
        grid=(TC_GRID,),
        in_specs=[
            pl.BlockSpec((ROWS_BLK, DIM), lambda i: (i, 0)),
            pl.BlockSpec((DIM, 1), lambda i: (0, 0)),
            pl.BlockSpec((1,), lambda i: (0,)),
        ],
        out_specs=pl.BlockSpec((ROWS_BLK, 1), lambda i: (i, 0)),
        out_shape=jax.ShapeDtypeStruct((VOCAB, 1), jnp.float32),
    )(table, W, b)


NC, NS = 2, 16
NW = NC * NS
RPW = BATCH // NW
IPW = RPW * L
CHUNK = 128
NCHUNK = IPW // CHUNK
GROUPS = RPW // 16


def _sc_body(t_hbm, bp_hbm, out_hbm, idx_v, vals_v, out_v, sem):
    wid = lax.axis_index("s") * NC + lax.axis_index("c")
    base = wid * RPW

    pltpu.sync_copy(bp_hbm.at[wid], idx_v)

    def _issue(c, carry):
        pltpu.make_async_copy(
            t_hbm.at[idx_v.at[c]], vals_v.at[pl.ds(c * CHUNK, CHUNK)], sem
        ).start()
        return carry

    lax.fori_loop(0, NCHUNK, _issue, 0)

    def _drain(c, carry):
        pltpu.make_async_copy(
            t_hbm.at[idx_v.at[c]], vals_v.at[pl.ds(c * CHUNK, CHUNK)], sem
        ).wait()
        return carry

    lax.fori_loop(0, NCHUNK, _drain, 0)

    def _reduce(g, carry):
        acc = jnp.zeros((16,), jnp.float32)
        for l in range(L):
            acc = acc + vals_v[pl.ds(l * RPW + g * 16, 16)]
        out_v[pl.ds(g * 16, 16)] = acc
        return carry

    lax.fori_loop(0, GROUPS, _reduce, 0)

    pltpu.sync_copy(out_v, out_hbm.at[pl.ds(base, RPW)])


@functools.cache
def _sc_gather():
    return functools.partial(
        pl.kernel,
        out_type=jax.ShapeDtypeStruct((BATCH,), jnp.float32),
        mesh=plsc.VectorSubcoreMesh(core_axis_name="c", subcore_axis_name="s"),
        scratch_types=[
            pltpu.VMEM((NCHUNK, CHUNK), jnp.int32),
            pltpu.VMEM((IPW,), jnp.float32),
            pltpu.VMEM((RPW,), jnp.float32),
            pltpu.SemaphoreType.DMA,
        ],
    )(_sc_body)


def kernel(batch, table, W, b):
    bp = (
        batch.astype(jnp.int32)
        .T.reshape(L, NW, RPW)
        .transpose(1, 0, 2)
        .reshape(NW, NCHUNK, CHUNK)
    )
    t = _matvec(table, W, b).reshape(VOCAB)
    return _sc_gather()(t, bp)


# speedup vs baseline: 5.5954x; 5.5954x over previous
import functools

import jax
import jax.numpy as jnp
from jax import lax
from jax.experimental import pallas as pl
from jax.experimental.pallas import tpu as pltpu
from jax.experimental.pallas import tpu_sc as plsc

VOCAB = 1_000_000
DIM = 64
BATCH = 16384
L = 50

COLS_BLK = 16384
TC_GRID = pl.cdiv(VOCAB, COLS_BLK)


def _matvec_body(x_ref, w_ref, b_ref, t_ref):
    t = jnp.sum(x_ref[...] * w_ref[...], axis=0)
    t_ref[...] = t + b_ref[...] * (1.0 / L)


def _matvec(table_t, W, b):
    return pl.pallas_call(
        _matvec_body,
        grid=(TC_GRID,),
        in_specs=[
            pl.BlockSpec((DIM, COLS_BLK), lambda i: (0, i)),
            pl.BlockSpec((DIM, 1), lambda i: (0, 0)),
            pl.BlockSpec((1,), lambda i: (0,)),
        ],
        out_specs=pl.BlockSpec((COLS_BLK,), lambda i: (i,)),
        out_shape=jax.ShapeDtypeStruct((VOCAB,), jnp.float32),
    )(table_t, W, b)


NC, NS = 2, 16
NW = NC * NS
RPW = BATCH // NW
IPW = RPW * L
CHUNK = 128
NCHUNK = IPW // CHUNK
GROUPS = RPW // 16


def _sc_body(t_hbm, bp_hbm, out_hbm, idx_v, vals_v, out_v, sem):
    wid = lax.axis_index("s") * NC + lax.axis_index("c")
    base = wid * RPW

    pltpu.sync_copy(bp_hbm.at[wid], idx_v)

    def _issue(c, carry):
        pltpu.make_async_copy(
            t_hbm.at[idx_v.at[c]], vals_v.at[pl.ds(c * CHUNK, CHUNK)], sem
        ).start()
        return carry

    lax.fori_loop(0, NCHUNK, _issue, 0)

    def _drain(c, carry):
        pltpu.make_async_copy(
            t_hbm.at[idx_v.at[c]], vals_v.at[pl.ds(c * CHUNK, CHUNK)], sem
        ).wait()
        return carry

    lax.fori_loop(0, NCHUNK, _drain, 0)

    def _reduce(g, carry):
        acc = jnp.zeros((16,), jnp.float32)
        for l in range(L):
            acc = acc + vals_v[pl.ds(l * RPW + g * 16, 16)]
        out_v[pl.ds(g * 16, 16)] = acc
        return carry

    lax.fori_loop(0, GROUPS, _reduce, 0)

    pltpu.sync_copy(out_v, out_hbm.at[pl.ds(base, RPW)])


@functools.cache
def _sc_gather():
    return functools.partial(
        pl.kernel,
        out_type=jax.ShapeDtypeStruct((BATCH,), jnp.float32),
        mesh=plsc.VectorSubcoreMesh(core_axis_name="c", subcore_axis_name="s"),
        scratch_types=[
            pltpu.VMEM((NCHUNK, CHUNK), jnp.int32),
            pltpu.VMEM((IPW,), jnp.float32),
            pltpu.VMEM((RPW,), jnp.float32),
            pltpu.SemaphoreType.DMA,
        ],
    )(_sc_body)


def kernel(batch, table, W, b):
    bp = (
        batch.astype(jnp.int32)
        .T.reshape(L, NW, RPW)
        .transpose(1, 0, 2)
        .reshape(NW, NCHUNK, CHUNK)
    )
    t = _matvec(table.T, W, b)
    return _sc_gather()(t, bp)


# speedup vs baseline: 6.5971x; 1.1790x over previous
import functools

import jax
import jax.numpy as jnp
from jax import lax
from jax.experimental import pallas as pl
from jax.experimental.pallas import tpu as pltpu
from jax.experimental.pallas import tpu_sc as plsc

VOCAB = 1_000_000
DIM = 64
BATCH = 16384
L = 50

COLS_BLK = 32768
TC_GRID = pl.cdiv(VOCAB, COLS_BLK)


def _matvec_body(x_ref, w_ref, b_ref, t_ref):
    t = jnp.sum(x_ref[...] * w_ref[...], axis=0)
    t_ref[...] = t + b_ref[...] * (1.0 / L)


def _matvec(table_t, W, b):
    return pl.pallas_call(
        _matvec_body,
        grid=(TC_GRID,),
        in_specs=[
            pl.BlockSpec((DIM, COLS_BLK), lambda i: (0, i)),
            pl.BlockSpec((DIM, 1), lambda i: (0, 0)),
            pl.BlockSpec((1,), lambda i: (0,)),
        ],
        out_specs=pl.BlockSpec((COLS_BLK,), lambda i: (i,)),
        out_shape=jax.ShapeDtypeStruct((VOCAB,), jnp.float32),
    )(table_t, W, b)


NC, NS = 2, 16
NW = NC * NS
RPW = BATCH // NW
IPW = RPW * L
CHUNK = 128
NCHUNK = IPW // CHUNK
CPL = RPW // CHUNK
GROUPS = RPW // 16


def _sc_body(t_hbm, bt_hbm, out_hbm, idx_v, vals_v, out_v, sem):
    wid = lax.axis_index("s") * NC + lax.axis_index("c")
    base = wid * RPW

    pltpu.sync_copy(bt_hbm.at[:, pl.ds(base, RPW)], idx_v)

    def _issue(c, carry):
        l = c // CPL
        cc = c % CPL
        pltpu.make_async_copy(
            t_hbm.at[idx_v.at[l, pl.ds(cc * CHUNK, CHUNK)]],
            vals_v.at[pl.ds(c * CHUNK, CHUNK)],
            sem,
        ).start()
        return carry

    lax.fori_loop(0, NCHUNK, _issue, 0)

    def _drain(c, carry):
        l = c // CPL
        cc = c % CPL
        pltpu.make_async_copy(
            t_hbm.at[idx_v.at[l, pl.ds(cc * CHUNK, CHUNK)]],
            vals_v.at[pl.ds(c * CHUNK, CHUNK)],
            sem,
        ).wait()
        return carry

    lax.fori_loop(0, NCHUNK, _drain, 0)

    def _reduce(g, carry):
        acc = jnp.zeros((16,), jnp.float32)
        for l in range(L):
            acc = acc + vals_v[pl.ds(l * RPW + g * 16, 16)]
        out_v[pl.ds(g * 16, 16)] = acc
        return carry

    lax.fori_loop(0, GROUPS, _reduce, 0)

    pltpu.sync_copy(out_v, out_hbm.at[pl.ds(base, RPW)])


@functools.cache
def _sc_gather():
    return functools.partial(
        pl.kernel,
        out_type=jax.ShapeDtypeStruct((BATCH,), jnp.float32),
        mesh=plsc.VectorSubcoreMesh(core_axis_name="c", subcore_axis_name="s"),
        scratch_types=[
            pltpu.VMEM((L, RPW), jnp.int32),
            pltpu.VMEM((IPW,), jnp.float32),
            pltpu.VMEM((RPW,), jnp.float32),
            pltpu.SemaphoreType.DMA,
        ],
    )(_sc_body)


def kernel(batch, table, W, b):
    bt = batch.astype(jnp.int32).T
    t = _matvec(table.T, W, b)
    return _sc_gather()(t, bt)
